# baseline (speedup 1.0000x reference)
import functools

import jax
import jax.numpy as jnp
from jax.experimental import pallas as pl
from jax.experimental.pallas import tpu as pltpu

_BOUNDARY_MARGIN = 2.0
_BN = 4096


def _fused_body(x_ref, cents_ref, W1cT_ref, W2cT_ref, b2T_ref,
                outT_ref, *, n_exp, hid, n_out):
    bn = x_ref.shape[0]
    xT = jnp.transpose(x_ref[...])
    x3T_full = xT[0:3, :]
    xfT_full = jnp.concatenate(
        [xT[3:, :].astype(jnp.bfloat16),
         jnp.ones((1, bn), dtype=jnp.bfloat16)], axis=0)
    cd2 = jnp.zeros((n_exp, bn), dtype=jnp.float32)
    for i in range(3):
        d = x3T_full[i:i + 1, :] - cents_ref[:, i:i + 1]
        cd2 = cd2 + d * d
    cd = jnp.sqrt(cd2)
    inv = 1.0 / (cd + 1e-8)
    mind = jnp.min(cd, axis=0, keepdims=True)
    inv = jnp.where(cd > _BOUNDARY_MARGIN * mind, 0.0, inv)
    wT = inv / jnp.sum(inv, axis=0, keepdims=True)

    xfT = xfT_full
    outT = jnp.dot(b2T_ref[...], wT,
                   preferred_element_type=jnp.float32)
    for e in range(n_exp):
        hT = jnp.dot(W1cT_ref[e * hid:(e + 1) * hid, :], xfT,
                     preferred_element_type=jnp.float32)
        hT = jnp.maximum(hT, 0.0)
        m2 = jnp.dot(W2cT_ref[e * n_out:(e + 1) * n_out, :], hT,
                     preferred_element_type=jnp.float32)
        outT = outT + m2 * wT[e:e + 1, :]
    outT_ref[...] = outT


def kernel(x, centroids, W1, b1, W2, b2):
    N = x.shape[0]
    E, D_IN, H = W1.shape
    OUT = W2.shape[-1]
    EH, EO = E * H, E * OUT

    W1cT = jnp.concatenate(
        [jnp.transpose(W1, (0, 2, 1)).reshape(EH, D_IN),
         b1.reshape(EH, 1)], axis=1).astype(jnp.bfloat16)
    W2cT = jnp.transpose(W2, (0, 2, 1)).reshape(EO, H)
    b2T = b2.T

    grid = (N // _BN,)
    body = functools.partial(_fused_body, n_exp=E, hid=H, n_out=OUT)
    outT = pl.pallas_call(
        body,
        grid=grid,
        in_specs=[
            pl.BlockSpec((_BN, 3 + D_IN), lambda i: (i, 0)),
            pl.BlockSpec((E, 3), lambda i: (0, 0)),
            pl.BlockSpec((EH, D_IN + 1), lambda i: (0, 0)),
            pl.BlockSpec((EO, H), lambda i: (0, 0)),
            pl.BlockSpec((OUT, E), lambda i: (0, 0)),
        ],
        out_specs=pl.BlockSpec((OUT, _BN), lambda i: (0, i)),
        out_shape=jax.ShapeDtypeStruct((OUT, N), jnp.float32),
        compiler_params=pltpu.CompilerParams(
            dimension_semantics=("parallel",)),
    )(x, centroids, W1cT, W2cT, b2T)
    return outT.T


# speedup vs baseline: 1.0823x; 1.0823x over previous
import functools

import jax
import jax.numpy as jnp
from jax.experimental import pallas as pl
from jax.experimental.pallas import tpu as pltpu

_BOUNDARY_MARGIN = 2.0
_BN = 4096


def _fused_body(x3T_ref, xfT_ref, cents_ref, W1cT_ref, W2cT_ref, b2T_ref,
                outT_ref, *, n_exp, hid, n_out):
    bn = x3T_ref.shape[1]
    cd2 = jnp.zeros((n_exp, bn), dtype=jnp.float32)
    for i in range(3):
        d = x3T_ref[i:i + 1, :] - cents_ref[:, i:i + 1]
        cd2 = cd2 + d * d
    cd = jnp.sqrt(cd2)
    inv = 1.0 / (cd + 1e-8)
    mind = jnp.min(cd, axis=0, keepdims=True)
    inv = jnp.where(cd > _BOUNDARY_MARGIN * mind, 0.0, inv)
    wT = inv / jnp.sum(inv, axis=0, keepdims=True)

    xfT = xfT_ref[...]
    outT = jnp.dot(b2T_ref[...], wT,
                   preferred_element_type=jnp.float32)
    for e in range(n_exp):
        hT = jnp.dot(W1cT_ref[e * hid:(e + 1) * hid, :], xfT,
                     preferred_element_type=jnp.float32)
        hT = jnp.maximum(hT, 0.0)
        m2 = jnp.dot(W2cT_ref[e * n_out:(e + 1) * n_out, :], hT,
                     preferred_element_type=jnp.float32)
        outT = outT + m2 * wT[e:e + 1, :]
    outT_ref[...] = outT


def kernel(x, centroids, W1, b1, W2, b2):
    N = x.shape[0]
    E, D_IN, H = W1.shape
    OUT = W2.shape[-1]
    EH, EO = E * H, E * OUT

    x3T = x[:, :3].T
    xfT = jnp.concatenate(
        [x[:, 3:].astype(jnp.bfloat16).T,
         jnp.ones((1, N), dtype=jnp.bfloat16)], axis=0)
    W1cT = jnp.concatenate(
        [jnp.transpose(W1, (0, 2, 1)).reshape(EH, D_IN),
         b1.reshape(EH, 1)], axis=1).astype(jnp.bfloat16)
    W2cT = jnp.transpose(W2, (0, 2, 1)).reshape(EO, H)
    b2T = b2.T

    grid = (N // _BN,)
    body = functools.partial(_fused_body, n_exp=E, hid=H, n_out=OUT)
    outT = pl.pallas_call(
        body,
        grid=grid,
        in_specs=[
            pl.BlockSpec((3, _BN), lambda i: (0, i)),
            pl.BlockSpec((D_IN + 1, _BN), lambda i: (0, i)),
            pl.BlockSpec((E, 3), lambda i: (0, 0)),
            pl.BlockSpec((EH, D_IN + 1), lambda i: (0, 0)),
            pl.BlockSpec((EO, H), lambda i: (0, 0)),
            pl.BlockSpec((OUT, E), lambda i: (0, 0)),
        ],
        out_specs=pl.BlockSpec((OUT, _BN), lambda i: (0, i)),
        out_shape=jax.ShapeDtypeStruct((OUT, N), jnp.float32),
        compiler_params=pltpu.CompilerParams(
            dimension_semantics=("parallel",)),
    )(x3T, xfT, centroids, W1cT, W2cT, b2T)
    return outT.T


# speedup vs baseline: 1.1032x; 1.0193x over previous
import functools

import jax
import jax.numpy as jnp
from jax.experimental import pallas as pl
from jax.experimental.pallas import tpu as pltpu

_BOUNDARY_MARGIN = 2.0
_BN = 8192


def _fused_body(x3T_ref, xfT_ref, cents_ref, W1cT_ref, W2cT_ref, b2T_ref,
                outT_ref, *, n_exp, hid, n_out):
    bn = x3T_ref.shape[1]
    cd2 = jnp.zeros((n_exp, bn), dtype=jnp.float32)
    for i in range(3):
        d = x3T_ref[i:i + 1, :] - cents_ref[:, i:i + 1]
        cd2 = cd2 + d * d
    cd = jnp.sqrt(cd2)
    inv = 1.0 / (cd + 1e-8)
    mind = jnp.min(cd, axis=0, keepdims=True)
    inv = jnp.where(cd > _BOUNDARY_MARGIN * mind, 0.0, inv)
    wT = inv / jnp.sum(inv, axis=0, keepdims=True)

    xfT = xfT_ref[...]
    outT = jnp.dot(b2T_ref[...], wT,
                   preferred_element_type=jnp.float32)
    for e in range(n_exp):
        hT = jnp.dot(W1cT_ref[e * hid:(e + 1) * hid, :], xfT,
                     preferred_element_type=jnp.float32)
        hT = jnp.maximum(hT, 0.0)
        m2 = jnp.dot(W2cT_ref[e * n_out:(e + 1) * n_out, :], hT,
                     preferred_element_type=jnp.float32)
        outT = outT + m2 * wT[e:e + 1, :]
    outT_ref[...] = outT


def kernel(x, centroids, W1, b1, W2, b2):
    N = x.shape[0]
    E, D_IN, H = W1.shape
    OUT = W2.shape[-1]
    EH, EO = E * H, E * OUT

    x3T = x[:, :3].T
    xfT = jnp.concatenate(
        [x[:, 3:].astype(jnp.bfloat16).T,
         jnp.ones((1, N), dtype=jnp.bfloat16)], axis=0)
    W1cT = jnp.concatenate(
        [jnp.transpose(W1, (0, 2, 1)).reshape(EH, D_IN),
         b1.reshape(EH, 1)], axis=1).astype(jnp.bfloat16)
    W2cT = jnp.transpose(W2, (0, 2, 1)).reshape(EO, H)
    b2T = b2.T

    grid = (N // _BN,)
    body = functools.partial(_fused_body, n_exp=E, hid=H, n_out=OUT)
    outT = pl.pallas_call(
        body,
        grid=grid,
        in_specs=[
            pl.BlockSpec((3, _BN), lambda i: (0, i)),
            pl.BlockSpec((D_IN + 1, _BN), lambda i: (0, i)),
            pl.BlockSpec((E, 3), lambda i: (0, 0)),
            pl.BlockSpec((EH, D_IN + 1), lambda i: (0, 0)),
            pl.BlockSpec((EO, H), lambda i: (0, 0)),
            pl.BlockSpec((OUT, E), lambda i: (0, 0)),
        ],
        out_specs=pl.BlockSpec((OUT, _BN), lambda i: (0, i)),
        out_shape=jax.ShapeDtypeStruct((OUT, N), jnp.float32),
        compiler_params=pltpu.CompilerParams(
            dimension_semantics=("parallel",)),
    )(x3T, xfT, centroids, W1cT, W2cT, b2T)
    return outT.T


# speedup vs baseline: 1.1059x; 1.0024x over previous
import functools

import jax
import jax.numpy as jnp
from jax.experimental import pallas as pl
from jax.experimental.pallas import tpu as pltpu

_BOUNDARY_MARGIN = 2.0
_BN = 16384


def _fused_body(x3T_ref, xfT_ref, cents_ref, W1cT_ref, W2cT_ref, b2T_ref,
                outT_ref, *, n_exp, hid, n_out):
    bn = x3T_ref.shape[1]
    cd2 = jnp.zeros((n_exp, bn), dtype=jnp.float32)
    for i in range(3):
        d = x3T_ref[i:i + 1, :] - cents_ref[:, i:i + 1]
        cd2 = cd2 + d * d
    cd = jnp.sqrt(cd2)
    inv = 1.0 / (cd + 1e-8)
    mind = jnp.min(cd, axis=0, keepdims=True)
    inv = jnp.where(cd > _BOUNDARY_MARGIN * mind, 0.0, inv)
    wT = inv / jnp.sum(inv, axis=0, keepdims=True)

    xfT = xfT_ref[...]
    outT = jnp.dot(b2T_ref[...], wT,
                   preferred_element_type=jnp.float32)
    for e in range(n_exp):
        hT = jnp.dot(W1cT_ref[e * hid:(e + 1) * hid, :], xfT,
                     preferred_element_type=jnp.float32)
        hT = jnp.maximum(hT, 0.0)
        m2 = jnp.dot(W2cT_ref[e * n_out:(e + 1) * n_out, :], hT,
                     preferred_element_type=jnp.float32)
        outT = outT + m2 * wT[e:e + 1, :]
    outT_ref[...] = outT


def kernel(x, centroids, W1, b1, W2, b2):
    N = x.shape[0]
    E, D_IN, H = W1.shape
    OUT = W2.shape[-1]
    EH, EO = E * H, E * OUT

    x3T = x[:, :3].T
    xfT = jnp.concatenate(
        [x[:, 3:].astype(jnp.bfloat16).T,
         jnp.ones((1, N), dtype=jnp.bfloat16)], axis=0)
    W1cT = jnp.concatenate(
        [jnp.transpose(W1, (0, 2, 1)).reshape(EH, D_IN),
         b1.reshape(EH, 1)], axis=1).astype(jnp.bfloat16)
    W2cT = jnp.transpose(W2, (0, 2, 1)).reshape(EO, H)
    b2T = b2.T

    grid = (N // _BN,)
    body = functools.partial(_fused_body, n_exp=E, hid=H, n_out=OUT)
    outT = pl.pallas_call(
        body,
        grid=grid,
        in_specs=[
            pl.BlockSpec((3, _BN), lambda i: (0, i)),
            pl.BlockSpec((D_IN + 1, _BN), lambda i: (0, i)),
            pl.BlockSpec((E, 3), lambda i: (0, 0)),
            pl.BlockSpec((EH, D_IN + 1), lambda i: (0, 0)),
            pl.BlockSpec((EO, H), lambda i: (0, 0)),
            pl.BlockSpec((OUT, E), lambda i: (0, 0)),
        ],
        out_specs=pl.BlockSpec((OUT, _BN), lambda i: (0, i)),
        out_shape=jax.ShapeDtypeStruct((OUT, N), jnp.float32),
        compiler_params=pltpu.CompilerParams(
            dimension_semantics=("parallel",)),
    )(x3T, xfT, centroids, W1cT, W2cT, b2T)
    return outT.T
